# Initial kernel scaffold; baseline (speedup 1.0000x reference)
import jax
import jax.numpy as jnp
from jax.experimental import pallas as pl


def kernel(image, feature_map, anchor_map, training, conv1_w, conv1_b, cls_w, cls_b, reg_w, reg_b):
    raise NotImplementedError("write your pallas kernel here")


# speedup vs baseline: 157.8794x; 157.8794x over previous
import jax
import jax.numpy as jnp
from jax.experimental import pallas as pl
from jax.experimental.pallas import tpu as pltpu

PRE_NMS = 6000
POST_NMS = 300
IOU_THR = 0.7
MIN_SIZE = 16.0
N_REAL = 22500
N_PAD = 23552
ROWS = 184
IMG_H = 800.0
IMG_W = 800.0
ONE_BITS = 0x3F800000


def _conv2d(x, w, b):
    y = jax.lax.conv_general_dilated(
        x[None], w, window_strides=(1, 1), padding='SAME',
        dimension_numbers=('NHWC', 'HWIO', 'NHWC'))[0]
    return y + b


def _nms_kernel(cnt0_ref, obj_ref, idx_ref,
                ay1_ref, ax1_ref, ay2_ref, ax2_ref,
                dy_ref, dx_ref, dh_ref, dw_ref,
                out_ref,
                y1_s, x1_s, y2_s, x2_s, area_s, pool_s):
    obj = obj_ref[...]
    idxv = idx_ref[...]

    hts = ay2_ref[...] - ay1_ref[...]
    wds = ax2_ref[...] - ax1_ref[...]
    cy = ay1_ref[...] + 0.5 * hts
    cx = ax1_ref[...] + 0.5 * wds
    pcy = dy_ref[...] * hts + cy
    pcx = dx_ref[...] * wds + cx
    ph = jnp.exp(dh_ref[...]) * hts
    pw = jnp.exp(dw_ref[...]) * wds
    y1 = jnp.maximum(pcy - 0.5 * ph, 0.0)
    x1 = jnp.maximum(pcx - 0.5 * pw, 0.0)
    y2 = jnp.minimum(pcy + 0.5 * ph, IMG_H)
    x2 = jnp.minimum(pcx + 0.5 * pw, IMG_W)
    valid = ((y2 - y1) >= MIN_SIZE) & ((x2 - x1) >= MIN_SIZE)
    y1_s[...] = y1
    x1_s[...] = x1
    y2_s[...] = y2
    x2_s[...] = x2
    area_s[...] = (y2 - y1) * (x2 - x1)

    def vstar_body(_, lohi):
        lo, hi = lohi
        mid = (lo + hi) // 2
        v = jax.lax.bitcast_convert_type(mid, jnp.float32)
        c = jnp.sum((obj >= v).astype(jnp.float32))
        big = c >= float(PRE_NMS)
        return (jnp.where(big, mid, lo), jnp.where(big, hi, mid))

    lo0 = jnp.int32(0)
    hi0 = jnp.int32(ONE_BITS + 1)
    lo, hi = jax.lax.fori_loop(0, 31, vstar_body, (lo0, hi0))
    vstar = jax.lax.bitcast_convert_type(lo, jnp.float32)

    gt = obj > vstar
    eq = obj == vstar
    need = float(PRE_NMS) - jnp.sum(gt.astype(jnp.float32))

    def tie_body(_, lohi):
        lo_i, hi_i = lohi
        mid = (lo_i + hi_i) // 2
        c = jnp.sum((eq & (idxv >= mid.astype(jnp.float32))).astype(jnp.float32))
        big = c >= need
        return (jnp.where(big, mid, lo_i), jnp.where(big, hi_i, mid))

    tlo, thi = jax.lax.fori_loop(0, 16, tie_body, (jnp.int32(0), jnp.int32(N_PAD)))
    cand = gt | (eq & (idxv >= tlo.astype(jnp.float32)))

    pool0 = cand & valid
    pool_s[...] = pool0.astype(jnp.float32)
    out_ref[...] = jnp.zeros((POST_NMS, 4), jnp.float32)

    def cond(state):
        count, nalive = state
        return (count < POST_NMS) & (nalive > 0.5)

    def body(state):
        count, _ = state
        pool = pool_s[...] > 0.5
        m = jnp.max(jnp.where(pool, obj, -jnp.inf))
        istar = jnp.max(jnp.where(pool & (obj == m), idxv, -1.0))
        sel = idxv == istar
        ninf = -jnp.inf
        by1 = jnp.max(jnp.where(sel, y1_s[...], ninf))
        bx1 = jnp.max(jnp.where(sel, x1_s[...], ninf))
        by2 = jnp.max(jnp.where(sel, y2_s[...], ninf))
        bx2 = jnp.max(jnp.where(sel, x2_s[...], ninf))
        barea = jnp.max(jnp.where(sel, area_s[...], ninf))
        lane = jax.lax.broadcasted_iota(jnp.int32, (1, 4), 1)
        row = jnp.where(lane == 0, by1,
                        jnp.where(lane == 1, bx1,
                                  jnp.where(lane == 2, by2, bx2)))
        out_ref[pl.ds(count, 1), :] = row
        yy1 = jnp.maximum(by1, y1_s[...])
        xx1 = jnp.maximum(bx1, x1_s[...])
        yy2 = jnp.minimum(by2, y2_s[...])
        xx2 = jnp.minimum(bx2, x2_s[...])
        inter = jnp.maximum(0.0, yy2 - yy1) * jnp.maximum(0.0, xx2 - xx1)
        iou = inter / (barea + area_s[...] - inter + 1e-9)
        keep_mask = jnp.logical_not(iou > IOU_THR)
        new_pool = pool_s[...] * keep_mask.astype(jnp.float32)
        pool_s[...] = new_pool
        return (count + 1, jnp.sum(new_pool))

    count0 = cnt0_ref[0, 0]
    nalive0 = jnp.sum(pool_s[...])
    jax.lax.while_loop(cond, body, (count0, nalive0))


def _pad_cols(v, fill):
    return jnp.concatenate(
        [v, jnp.full((N_PAD - N_REAL,), fill, v.dtype)]).reshape(ROWS, 128)


def kernel(image, feature_map, anchor_map, training, conv1_w, conv1_b, cls_w, cls_b, reg_w, reg_b):
    del image
    y = jax.nn.relu(_conv2d(feature_map, conv1_w, conv1_b))
    scores = jax.nn.softmax(_conv2d(y, cls_w, cls_b), axis=-1)
    bbox_regressions = _conv2d(y, reg_w, reg_b)
    objectness = scores.reshape(N_REAL)
    box_deltas = bbox_regressions.reshape(N_REAL, 4)

    objp = _pad_cols(objectness, -1.0)
    idxp = _pad_cols(jnp.arange(N_REAL, dtype=jnp.float32), -1.0)
    ay1 = _pad_cols(anchor_map[:, 0], 0.0)
    ax1 = _pad_cols(anchor_map[:, 1], 0.0)
    ay2 = _pad_cols(anchor_map[:, 2], 0.0)
    ax2 = _pad_cols(anchor_map[:, 3], 0.0)
    dy = _pad_cols(box_deltas[:, 0], 0.0)
    dx = _pad_cols(box_deltas[:, 1], 0.0)
    dh = _pad_cols(box_deltas[:, 2], 0.0)
    dw = _pad_cols(box_deltas[:, 3], 0.0)
    cnt0 = jnp.asarray(training, jnp.int32).reshape(1, 1)

    out = pl.pallas_call(
        _nms_kernel,
        out_shape=jax.ShapeDtypeStruct((POST_NMS, 4), jnp.float32),
        in_specs=[pl.BlockSpec(memory_space=pltpu.SMEM)] +
                 [pl.BlockSpec(memory_space=pltpu.VMEM)] * 10,
        out_specs=pl.BlockSpec(memory_space=pltpu.VMEM),
        scratch_shapes=[pltpu.VMEM((ROWS, 128), jnp.float32)] * 6,
    )(cnt0, objp, idxp, ay1, ax1, ay2, ax2, dy, dx, dh, dw)
    return (scores, box_deltas, out)


# speedup vs baseline: 185.2250x; 1.1732x over previous
import jax
import jax.numpy as jnp
from jax.experimental import pallas as pl
from jax.experimental.pallas import tpu as pltpu

PRE_NMS = 6000
POST_NMS = 300
IOU_THR = 0.7
MIN_SIZE = 16.0
N_REAL = 22500
N_PAD = 23552
ROWS = 184
IMG_H = 800.0
IMG_W = 800.0
ONE_BITS = 0x3F800000


def _conv2d(x, w, b):
    y = jax.lax.conv_general_dilated(
        x[None], w, window_strides=(1, 1), padding='SAME',
        dimension_numbers=('NHWC', 'HWIO', 'NHWC'))[0]
    return y + b


def _nms_kernel(cnt0_ref, obj_ref, idx_ref,
                ay1_ref, ax1_ref, ay2_ref, ax2_ref,
                dy_ref, dx_ref, dh_ref, dw_ref,
                out_ref,
                y1_s, x1_s, y2_s, x2_s, area_s, pool_s):
    obj = obj_ref[...]
    idxv = idx_ref[...]

    hts = ay2_ref[...] - ay1_ref[...]
    wds = ax2_ref[...] - ax1_ref[...]
    cy = ay1_ref[...] + 0.5 * hts
    cx = ax1_ref[...] + 0.5 * wds
    pcy = dy_ref[...] * hts + cy
    pcx = dx_ref[...] * wds + cx
    ph = jnp.exp(dh_ref[...]) * hts
    pw = jnp.exp(dw_ref[...]) * wds
    y1 = jnp.maximum(pcy - 0.5 * ph, 0.0)
    x1 = jnp.maximum(pcx - 0.5 * pw, 0.0)
    y2 = jnp.minimum(pcy + 0.5 * ph, IMG_H)
    x2 = jnp.minimum(pcx + 0.5 * pw, IMG_W)
    valid = ((y2 - y1) >= MIN_SIZE) & ((x2 - x1) >= MIN_SIZE)
    y1_s[...] = y1
    x1_s[...] = x1
    y2_s[...] = y2
    x2_s[...] = x2
    area_s[...] = (y2 - y1) * (x2 - x1)

    def vstar_body(_, lohi):
        lo, hi = lohi
        mid = (lo + hi) // 2
        v = jax.lax.bitcast_convert_type(mid, jnp.float32)
        c = jnp.sum((obj >= v).astype(jnp.float32))
        big = c >= float(PRE_NMS)
        return (jnp.where(big, mid, lo), jnp.where(big, hi, mid))

    lo0 = jnp.int32(0)
    hi0 = jnp.int32(ONE_BITS + 1)
    lo, hi = jax.lax.fori_loop(0, 31, vstar_body, (lo0, hi0))
    vstar = jax.lax.bitcast_convert_type(lo, jnp.float32)

    gt = obj > vstar
    eq = obj == vstar
    need = float(PRE_NMS) - jnp.sum(gt.astype(jnp.float32))

    def tie_body(_, lohi):
        lo_i, hi_i = lohi
        mid = (lo_i + hi_i) // 2
        c = jnp.sum((eq & (idxv >= mid.astype(jnp.float32))).astype(jnp.float32))
        big = c >= need
        return (jnp.where(big, mid, lo_i), jnp.where(big, hi_i, mid))

    tlo, thi = jax.lax.fori_loop(0, 16, tie_body, (jnp.int32(0), jnp.int32(N_PAD)))
    cand = gt | (eq & (idxv >= tlo.astype(jnp.float32)))

    ninf = -jnp.inf
    objm0 = jnp.where(cand & valid, obj, ninf)
    pool_s[...] = objm0
    out_ref[...] = jnp.zeros((POST_NMS, 4), jnp.float32)

    def cond(state):
        count, m = state
        return (count < POST_NMS) & (m > ninf)

    def body(state):
        count, m = state
        objm = pool_s[...]
        istar = jnp.max(jnp.where(objm == m, idxv, -1.0))
        ii = istar.astype(jnp.int32)
        r = ii // 128
        c = ii % 128
        lane128 = jax.lax.broadcasted_iota(jnp.int32, (1, 128), 1)
        onehot = lane128 == c
        by1 = jnp.max(jnp.where(onehot, y1_s[pl.ds(r, 1), :], ninf))
        bx1 = jnp.max(jnp.where(onehot, x1_s[pl.ds(r, 1), :], ninf))
        by2 = jnp.max(jnp.where(onehot, y2_s[pl.ds(r, 1), :], ninf))
        bx2 = jnp.max(jnp.where(onehot, x2_s[pl.ds(r, 1), :], ninf))
        barea = (by2 - by1) * (bx2 - bx1)
        lane = jax.lax.broadcasted_iota(jnp.int32, (1, 4), 1)
        row = jnp.where(lane == 0, by1,
                        jnp.where(lane == 1, bx1,
                                  jnp.where(lane == 2, by2, bx2)))
        out_ref[pl.ds(count, 1), :] = row
        yy1 = jnp.maximum(by1, y1_s[...])
        xx1 = jnp.maximum(bx1, x1_s[...])
        yy2 = jnp.minimum(by2, y2_s[...])
        xx2 = jnp.minimum(bx2, x2_s[...])
        inter = jnp.maximum(0.0, yy2 - yy1) * jnp.maximum(0.0, xx2 - xx1)
        iou = inter / (barea + area_s[...] - inter + 1e-9)
        new_objm = jnp.where(iou > IOU_THR, ninf, objm)
        pool_s[...] = new_objm
        return (count + 1, jnp.max(new_objm))

    count0 = cnt0_ref[0, 0]
    m0 = jnp.max(objm0)
    jax.lax.while_loop(cond, body, (count0, m0))


def _pad_cols(v, fill):
    return jnp.concatenate(
        [v, jnp.full((N_PAD - N_REAL,), fill, v.dtype)]).reshape(ROWS, 128)


def kernel(image, feature_map, anchor_map, training, conv1_w, conv1_b, cls_w, cls_b, reg_w, reg_b):
    del image
    y = jax.nn.relu(_conv2d(feature_map, conv1_w, conv1_b))
    scores = jax.nn.softmax(_conv2d(y, cls_w, cls_b), axis=-1)
    bbox_regressions = _conv2d(y, reg_w, reg_b)
    objectness = scores.reshape(N_REAL)
    box_deltas = bbox_regressions.reshape(N_REAL, 4)

    objp = _pad_cols(objectness, -1.0)
    idxp = _pad_cols(jnp.arange(N_REAL, dtype=jnp.float32), -1.0)
    ay1 = _pad_cols(anchor_map[:, 0], 0.0)
    ax1 = _pad_cols(anchor_map[:, 1], 0.0)
    ay2 = _pad_cols(anchor_map[:, 2], 0.0)
    ax2 = _pad_cols(anchor_map[:, 3], 0.0)
    dy = _pad_cols(box_deltas[:, 0], 0.0)
    dx = _pad_cols(box_deltas[:, 1], 0.0)
    dh = _pad_cols(box_deltas[:, 2], 0.0)
    dw = _pad_cols(box_deltas[:, 3], 0.0)
    cnt0 = jnp.asarray(training, jnp.int32).reshape(1, 1)

    out = pl.pallas_call(
        _nms_kernel,
        out_shape=jax.ShapeDtypeStruct((POST_NMS, 4), jnp.float32),
        in_specs=[pl.BlockSpec(memory_space=pltpu.SMEM)] +
                 [pl.BlockSpec(memory_space=pltpu.VMEM)] * 10,
        out_specs=pl.BlockSpec(memory_space=pltpu.VMEM),
        scratch_shapes=[pltpu.VMEM((ROWS, 128), jnp.float32)] * 6,
    )(cnt0, objp, idxp, ay1, ax1, ay2, ax2, dy, dx, dh, dw)
    return (scores, box_deltas, out)
